# Initial kernel scaffold; baseline (speedup 1.0000x reference)
import jax
import jax.numpy as jnp
from jax.experimental import pallas as pl


def kernel(x, dummy, edge_index, W1_0, W1_1, b1, W2_0, W2_1, b2, W3_0, W3_1, b3, Wl, bl):
    raise NotImplementedError("write your pallas kernel here")


# speedup vs baseline: 26.1963x; 26.1963x over previous
import functools

import jax
import jax.numpy as jnp
from jax import lax
from jax.experimental import pallas as pl
from jax.experimental.pallas import tpu as pltpu
from jax.experimental.pallas import tpu_sc as plsc

B, C, N, E = 4, 128, 10000, 160000
H = 16
NCLS = 10
BN = B * N
TRASH = BN
ACCN = 40960
DEG_TRASH = N
DEGP = 10240
EP = 163840
EPR = EP // 128
NSC, NTEC = 2, 16
NW = NSC * NTEC
W_CH = EPR // NW
W_EXP = EPR // 8
ACC_T = ACCN // NTEC
DEG_T = DEGP // NTEC

BLK = 400
NBLK = BN // BLK
BPG = N // BLK

@functools.cache
def _build_sc_pre():
  mesh = plsc.VectorSubcoreMesh(
      core_axis_name="c", subcore_axis_name="s",
      num_cores=NSC, num_subcores=NTEC)

  @functools.partial(
      pl.kernel,
      out_type=(
          jax.ShapeDtypeStruct((NSC, DEGP), jnp.float32),
          jax.ShapeDtypeStruct((B, EPR, 128), jnp.int32),
          jax.ShapeDtypeStruct((B, EPR, 128), jnp.int32),
      ),
      mesh=mesh,
      compiler_params=pltpu.CompilerParams(use_tc_tiling_on_sc=False),
      scratch_types=[
          pltpu.VMEM((W_EXP, 128), jnp.int32),
          pltpu.VMEM((W_EXP, 128), jnp.int32),
          pltpu.VMEM((W_EXP, 128), jnp.int32),
          pltpu.VMEM((W_EXP, 128), jnp.int32),
          pltpu.VMEM((W_CH, 128), jnp.int32),
          pltpu.VMEM((128,), jnp.float32),
          pltpu.VMEM((DEG_T,), jnp.float32),
          pltpu.VMEM_SHARED((DEGP,), jnp.float32),
      ],
  )
  def _sc_pre(row_hbm, col_hbm, deg_out, rg_out, cg_out,
              rbuf, cbuf, rgbuf, cgbuf, dix, ones_v, zeros_v, deg_acc):
    c = lax.axis_index("c")
    s = lax.axis_index("s")
    gwid = c * NTEC + s

    def _fill(i, _):
      ones_v[pl.ds(i * 16, 16)] = jnp.full((16,), 1.0, jnp.float32)
      return 0
    lax.fori_loop(0, 8, _fill, 0)

    def _zfill(i, _):
      zeros_v[pl.ds(i * 16, 16)] = jnp.zeros((16,), jnp.float32)
      return 0
    lax.fori_loop(0, DEG_T // 16, _zfill, 0)

    pltpu.sync_copy(zeros_v, deg_acc.at[pl.ds(s * DEG_T, DEG_T)])
    plsc.subcore_barrier()

    pltpu.sync_copy(row_hbm.at[pl.ds(gwid * W_CH, W_CH)], rbuf.at[pl.ds(0, W_CH)])
    pltpu.sync_copy(col_hbm.at[pl.ds(gwid * W_CH, W_CH)], cbuf.at[pl.ds(0, W_CH)])

    def _deg_chunk(j, _):
      for l in range(8):
        rv = rbuf[j, pl.ds(l * 16, 16)]
        cv = cbuf[j, pl.ds(l * 16, 16)]
        dix[j, pl.ds(l * 16, 16)] = jnp.where(
            rv == cv, jnp.full((16,), DEG_TRASH, jnp.int32), rv)
      pltpu.sync_copy(ones_v, deg_acc.at[dix.at[j]], add=True)
      return 0
    lax.fori_loop(0, W_CH, _deg_chunk, 0)

    plsc.subcore_barrier()
    pltpu.sync_copy(deg_acc.at[pl.ds(s * DEG_T, DEG_T)],
                    deg_out.at[c, pl.ds(s * DEG_T, DEG_T)])

    b = gwid // 8
    w8 = gwid % 8
    bofs = b * N
    pltpu.sync_copy(row_hbm.at[pl.ds(w8 * W_EXP, W_EXP)], rbuf)
    pltpu.sync_copy(col_hbm.at[pl.ds(w8 * W_EXP, W_EXP)], cbuf)

    def _exp_chunk(j, _):
      for l in range(8):
        rv = rbuf[j, pl.ds(l * 16, 16)]
        cv = cbuf[j, pl.ds(l * 16, 16)]
        rgbuf[j, pl.ds(l * 16, 16)] = rv + bofs
        cgbuf[j, pl.ds(l * 16, 16)] = jnp.where(
            rv == cv, jnp.full((16,), TRASH, jnp.int32), cv + bofs)
      return 0
    lax.fori_loop(0, W_EXP, _exp_chunk, 0)

    pltpu.sync_copy(rgbuf, rg_out.at[b, pl.ds(w8 * W_EXP, W_EXP)])
    pltpu.sync_copy(cgbuf, cg_out.at[b, pl.ds(w8 * W_EXP, W_EXP)])

  return _sc_pre


@functools.cache
def _build_sc_prop():
  mesh = plsc.VectorSubcoreMesh(
      core_axis_name="c", subcore_axis_name="s",
      num_cores=NSC, num_subcores=NTEC)

  @functools.partial(
      pl.kernel,
      out_type=jax.ShapeDtypeStruct((NSC, ACCN, H), jnp.float32),
      mesh=mesh,
      compiler_params=pltpu.CompilerParams(use_tc_tiling_on_sc=False),
      scratch_types=[
          pltpu.VMEM((W_EXP, 128), jnp.int32),
          pltpu.VMEM((W_EXP, 128), jnp.int32),
          pltpu.VMEM((128, H), jnp.float32),
          pltpu.VMEM((128, H), jnp.float32),
          pltpu.VMEM_SHARED((ACCN, H), jnp.float32),
          pltpu.SemaphoreType.DMA,
      ],
  )
  def _sc_prop(u_hbm, rg_hbm, cg_hbm, out_hbm, rix, cix, rows_v, zeros_v, acc, sem):
    c = lax.axis_index("c")
    s = lax.axis_index("s")
    gwid = c * NTEC + s
    b = gwid // 8
    w8 = gwid % 8

    def _zfill(i, _):
      zeros_v[i, :] = jnp.zeros((H,), jnp.float32)
      return 0
    lax.fori_loop(0, 128, _zfill, 0)

    def _zacc(k, _):
      pltpu.sync_copy(zeros_v, acc.at[pl.ds(s * ACC_T + k * 128, 128)])
      return 0
    lax.fori_loop(0, ACC_T // 128, _zacc, 0)
    plsc.subcore_barrier()

    pltpu.sync_copy(rg_hbm.at[b, pl.ds(w8 * W_EXP, W_EXP)], rix)
    pltpu.sync_copy(cg_hbm.at[b, pl.ds(w8 * W_EXP, W_EXP)], cix)

    def _edge_chunk(j, _):
      pltpu.async_copy(u_hbm.at[rix.at[j]], rows_v, sem).wait()
      pltpu.sync_copy(rows_v, acc.at[cix.at[j]], add=True)
      return 0
    lax.fori_loop(0, W_EXP, _edge_chunk, 0)

    plsc.subcore_barrier()
    pltpu.sync_copy(acc.at[pl.ds(s * ACC_T, ACC_T)],
                    out_hbm.at[c, pl.ds(s * ACC_T, ACC_T)])

  return _sc_prop


def _dinv_of(deg_ref):
    d = deg_ref[0, 0, 0] + deg_ref[1, 0, 0]
    return jnp.where(d > 0, lax.rsqrt(jnp.where(d > 0, d, 1.0)), 0.0)


def _elu(v):
    return jnp.where(v > 0, v, jnp.exp(jnp.minimum(v, 0.0)) - 1.0)


def _tc1_body(x_ref, w_ref, deg_ref, a_ref, up_ref):
    xb = x_ref[0, :, 0, 0, :]
    t = lax.dot_general(xb, w_ref[...], (((0,), (0,)), ((), ())),
                        preferred_element_type=jnp.float32)
    dinv = _dinv_of(deg_ref)
    a_ref[...] = t[:, :H]
    up_ref[...] = t[:, H:] * dinv[:, None]


def _tc_mid_body(a_ref, s_ref, deg_ref, w_ref, b_ref, a_out, up_out):
    dinv = _dinv_of(deg_ref)
    S = s_ref[0] + s_ref[1]
    h = _elu(a_ref[...] - dinv[:, None] * S + b_ref[...])
    t = jnp.dot(h, w_ref[...], preferred_element_type=jnp.float32)
    a_out[...] = t[:, :H]
    up_out[...] = t[:, H:] * dinv[:, None]


def _tc4_body(a_ref, s_ref, deg_ref, b_ref, wl_ref, bl_ref, o_ref, pool):
    i = pl.program_id(0)

    @pl.when(i == 0)
    def _():
        pool[...] = jnp.zeros((8, 128), jnp.float32)

    dinv = _dinv_of(deg_ref)
    S = s_ref[0] + s_ref[1]
    h = _elu(a_ref[...] - dinv[:, None] * S + b_ref[...])
    g = i // BPG
    ssum = jnp.sum(h, axis=0, keepdims=True)
    srow = jnp.pad(ssum, ((0, 0), (0, 128 - H)))
    pool[pl.ds(g, 1), :] = pool[pl.ds(g, 1), :] + srow

    @pl.when(i == NBLK - 1)
    def _():
        pooled = pool[0:B, 0:H] / float(N)
        out = jnp.dot(pooled, wl_ref[...],
                      preferred_element_type=jnp.float32) + bl_ref[...]
        m = jnp.max(out, axis=1, keepdims=True)
        z = out - m
        lse = jnp.log(jnp.sum(jnp.exp(z), axis=1, keepdims=True))
        o_ref[...] = z - lse


_deg_spec = pl.BlockSpec((NSC, 1, 1, BLK), lambda i: (0, i % BPG, 0, 0))
_nf_spec = pl.BlockSpec((BLK, H), lambda i: (i, 0))
_acc_spec = pl.BlockSpec((2, BLK, H), lambda i: (0, i, 0))
_b_spec = pl.BlockSpec((1, H), lambda i: (0, 0))

_tc1 = pl.pallas_call(
    _tc1_body,
    grid=(NBLK,),
    in_specs=[
        pl.BlockSpec((1, C, 1, 1, BLK), lambda i: (i // BPG, 0, i % BPG, 0, 0)),
        pl.BlockSpec((C, 2 * H), lambda i: (0, 0)),
        _deg_spec,
    ],
    out_specs=[_nf_spec, _nf_spec],
    out_shape=[
        jax.ShapeDtypeStruct((BN, H), jnp.float32),
        jax.ShapeDtypeStruct((BN, H), jnp.float32),
    ],
)

_tc_mid = pl.pallas_call(
    _tc_mid_body,
    grid=(NBLK,),
    in_specs=[
        _nf_spec,
        _acc_spec,
        _deg_spec,
        pl.BlockSpec((H, 2 * H), lambda i: (0, 0)),
        _b_spec,
    ],
    out_specs=[_nf_spec, _nf_spec],
    out_shape=[
        jax.ShapeDtypeStruct((BN, H), jnp.float32),
        jax.ShapeDtypeStruct((BN, H), jnp.float32),
    ],
)

_tc4 = pl.pallas_call(
    _tc4_body,
    grid=(NBLK,),
    in_specs=[
        _nf_spec,
        _acc_spec,
        _deg_spec,
        _b_spec,
        pl.BlockSpec((H, NCLS), lambda i: (0, 0)),
        pl.BlockSpec((1, NCLS), lambda i: (0, 0)),
    ],
    out_specs=pl.BlockSpec((B, NCLS), lambda i: (0, 0)),
    out_shape=jax.ShapeDtypeStruct((B, NCLS), jnp.float32),
    scratch_shapes=[pltpu.VMEM((8, 128), jnp.float32)],
)


def kernel(x, dummy, edge_index, W1_0, W1_1, b1, W2_0, W2_1, b2,
           W3_0, W3_1, b3, Wl, bl):
    row = jnp.pad(edge_index[0], (0, EP - E)).reshape(EPR, 128)
    col = jnp.pad(edge_index[1], (0, EP - E)).reshape(EPR, 128)

    deg, rg, cg = _build_sc_pre()(row, col)
    _sc_prop = _build_sc_prop()

    w1 = jnp.concatenate([W1_0, W1_1], axis=1)
    w2 = jnp.concatenate([W2_0, W2_1], axis=1)
    w3 = jnp.concatenate([W3_0, W3_1], axis=1)

    x5 = x.reshape(B, C, BPG, 1, BLK)
    deg4 = deg[:, :N].reshape(NSC, BPG, 1, BLK)

    a1, u1 = _tc1(x5, w1, deg4)
    s1 = _sc_prop(u1, rg, cg)
    a2, u2 = _tc_mid(a1, s1, deg4, w2, b1.reshape(1, H))
    s2 = _sc_prop(u2, rg, cg)
    a3, u3 = _tc_mid(a2, s2, deg4, w3, b2.reshape(1, H))
    s3 = _sc_prop(u3, rg, cg)
    return _tc4(a3, s3, deg4, b3.reshape(1, H), Wl, bl.reshape(1, NCLS))


# speedup vs baseline: 65.6217x; 2.5050x over previous
import functools

import jax
import jax.numpy as jnp
from jax import lax
from jax.experimental import pallas as pl
from jax.experimental.pallas import tpu as pltpu
from jax.experimental.pallas import tpu_sc as plsc

B, C, N, E = 4, 128, 10000, 160000
H = 16
NCLS = 10
BN = B * N
TRASH = BN
LACC = 20480
LTRASH = 2 * N
DEG_TRASH = N
DEGP = 10240
EP = 163840
EPR = EP // 128
NSC, NTEC = 2, 16
NW = NSC * NTEC
W_CH = EPR // NW
W_EXP = EPR // 8
DEG_T = DEGP // NTEC
LACC_T = LACC // NTEC
OUT_T = (2 * N) // NTEC
RING = 8

BLK = 400
NBLK = BN // BLK
BPG = N // BLK

@functools.cache
def _build_sc_pre():
  mesh = plsc.VectorSubcoreMesh(
      core_axis_name="c", subcore_axis_name="s",
      num_cores=NSC, num_subcores=NTEC)

  @functools.partial(
      pl.kernel,
      out_type=(
          jax.ShapeDtypeStruct((NSC, DEGP), jnp.float32),
          jax.ShapeDtypeStruct((B, EPR, 128), jnp.int32),
          jax.ShapeDtypeStruct((B, EPR, 128), jnp.int32),
      ),
      mesh=mesh,
      compiler_params=pltpu.CompilerParams(use_tc_tiling_on_sc=False),
      scratch_types=[
          pltpu.VMEM((W_EXP, 128), jnp.int32),
          pltpu.VMEM((W_EXP, 128), jnp.int32),
          pltpu.VMEM((W_EXP, 128), jnp.int32),
          pltpu.VMEM((W_EXP, 128), jnp.int32),
          pltpu.VMEM((W_CH, 128), jnp.int32),
          pltpu.VMEM((128,), jnp.float32),
          pltpu.VMEM((DEG_T,), jnp.float32),
          pltpu.VMEM_SHARED((DEGP,), jnp.float32),
          pltpu.SemaphoreType.DMA,
      ],
  )
  def _sc_pre(row_hbm, col_hbm, deg_out, rg_out, cg_out,
              rbuf, cbuf, rgbuf, cgbuf, dix, ones_v, zeros_v, deg_acc, dsem):
    c = lax.axis_index("c")
    s = lax.axis_index("s")
    gwid = c * NTEC + s

    def _fill(i, _):
      ones_v[pl.ds(i * 16, 16)] = jnp.full((16,), 1.0, jnp.float32)
      return 0
    lax.fori_loop(0, 8, _fill, 0)

    def _zfill(i, _):
      zeros_v[pl.ds(i * 16, 16)] = jnp.zeros((16,), jnp.float32)
      return 0
    lax.fori_loop(0, DEG_T // 16, _zfill, 0)

    pltpu.sync_copy(zeros_v, deg_acc.at[pl.ds(s * DEG_T, DEG_T)])
    plsc.subcore_barrier()

    pltpu.sync_copy(row_hbm.at[pl.ds(gwid * W_CH, W_CH)], rbuf.at[pl.ds(0, W_CH)])
    pltpu.sync_copy(col_hbm.at[pl.ds(gwid * W_CH, W_CH)], cbuf.at[pl.ds(0, W_CH)])

    def _deg_chunk(j, _):
      for l in range(8):
        rv = rbuf[j, pl.ds(l * 16, 16)]
        cv = cbuf[j, pl.ds(l * 16, 16)]
        dix[j, pl.ds(l * 16, 16)] = jnp.where(
            rv == cv, jnp.full((16,), DEG_TRASH, jnp.int32), rv)
      pltpu.async_copy(ones_v, deg_acc.at[dix.at[j]], dsem, add=True)
      return 0
    lax.fori_loop(0, W_CH, _deg_chunk, 0)

    def _deg_drain(j, _):
      pltpu.make_async_copy(ones_v, deg_acc.at[dix.at[j]], dsem).wait()
      return 0
    lax.fori_loop(0, W_CH, _deg_drain, 0)

    plsc.subcore_barrier()
    pltpu.sync_copy(deg_acc.at[pl.ds(s * DEG_T, DEG_T)],
                    deg_out.at[c, pl.ds(s * DEG_T, DEG_T)])

    b = gwid // 8
    w8 = gwid % 8
    bofs = b * N
    lofs = (b % 2) * N
    pltpu.sync_copy(row_hbm.at[pl.ds(w8 * W_EXP, W_EXP)], rbuf)
    pltpu.sync_copy(col_hbm.at[pl.ds(w8 * W_EXP, W_EXP)], cbuf)

    def _exp_chunk(j, _):
      for l in range(8):
        rv = rbuf[j, pl.ds(l * 16, 16)]
        cv = cbuf[j, pl.ds(l * 16, 16)]
        rgbuf[j, pl.ds(l * 16, 16)] = rv + bofs
        cgbuf[j, pl.ds(l * 16, 16)] = jnp.where(
            rv == cv, jnp.full((16,), LTRASH, jnp.int32), cv + lofs)
      return 0
    lax.fori_loop(0, W_EXP, _exp_chunk, 0)

    pltpu.sync_copy(rgbuf, rg_out.at[b, pl.ds(w8 * W_EXP, W_EXP)])
    pltpu.sync_copy(cgbuf, cg_out.at[b, pl.ds(w8 * W_EXP, W_EXP)])

  return _sc_pre


@functools.cache
def _build_sc_prop():
  mesh = plsc.VectorSubcoreMesh(
      core_axis_name="c", subcore_axis_name="s",
      num_cores=NSC, num_subcores=NTEC)

  @functools.partial(
      pl.kernel,
      out_type=jax.ShapeDtypeStruct((BN, H), jnp.float32),
      mesh=mesh,
      compiler_params=pltpu.CompilerParams(use_tc_tiling_on_sc=False),
      scratch_types=[
          pltpu.VMEM((W_EXP, 128), jnp.int32),
          pltpu.VMEM((W_EXP, 128), jnp.int32),
          pltpu.VMEM((RING * 128, H), jnp.float32),
          pltpu.VMEM((128, H), jnp.float32),
          pltpu.VMEM_SHARED((LACC, H), jnp.float32),
          [pltpu.SemaphoreType.DMA] * RING,
          [pltpu.SemaphoreType.DMA] * RING,
      ],
  )
  def _sc_prop(u_hbm, rg_hbm, cg_hbm, out_hbm,
               rix, cix, rows_v, zeros_v, acc, gsem, ssem):
    c = lax.axis_index("c")
    s = lax.axis_index("s")
    gwid = c * NTEC + s
    b = gwid // 8
    w8 = gwid % 8

    def _zfill(i, _):
      zeros_v[i, :] = jnp.zeros((H,), jnp.float32)
      return 0
    lax.fori_loop(0, 128, _zfill, 0)

    def _zacc(k, _):
      pltpu.sync_copy(zeros_v, acc.at[pl.ds(s * LACC_T + k * 128, 128)])
      return 0
    lax.fori_loop(0, LACC_T // 128, _zacc, 0)

    pltpu.sync_copy(rg_hbm.at[b, pl.ds(w8 * W_EXP, W_EXP)], rix)
    pltpu.sync_copy(cg_hbm.at[b, pl.ds(w8 * W_EXP, W_EXP)], cix)
    plsc.subcore_barrier()

    def _slot(r):
      return rows_v.at[pl.ds(r * 128, 128)]

    for r in range(RING):
      pltpu.async_copy(u_hbm.at[rix.at[r]], _slot(r), gsem[r])

    def _round(k, _):
      js = [k * RING + r for r in range(RING)]
      for r in range(RING):
        pltpu.make_async_copy(u_hbm.at[rix.at[js[r]]], _slot(r), gsem[r]).wait()
        pltpu.async_copy(_slot(r), acc.at[cix.at[js[r]]], ssem[r], add=True)
      for r in range(RING):
        pltpu.make_async_copy(_slot(r), acc.at[cix.at[js[r]]], ssem[r]).wait()

        @pl.when(k < W_EXP // RING - 1)
        def _():
          pltpu.async_copy(u_hbm.at[rix.at[js[r] + RING]], _slot(r), gsem[r])
      return 0
    lax.fori_loop(0, W_EXP // RING, _round, 0)

    plsc.subcore_barrier()
    pltpu.sync_copy(acc.at[pl.ds(s * OUT_T, OUT_T)],
                    out_hbm.at[pl.ds(c * 2 * N + s * OUT_T, OUT_T)])

  return _sc_prop


def _dinv_of(deg_ref):
    d = deg_ref[0, 0] + deg_ref[1, 0]
    return jnp.where(d > 0, lax.rsqrt(jnp.where(d > 0, d, 1.0)), 0.0)


def _elu(v):
    return jnp.where(v > 0, v, jnp.exp(jnp.minimum(v, 0.0)) - 1.0)


def _tc1_body(x_ref, w_ref, deg_ref, a_ref, up_ref):
    xb = x_ref[0]
    t = lax.dot_general(xb, w_ref[...], (((0,), (0,)), ((), ())),
                        preferred_element_type=jnp.float32)
    dinv = _dinv_of(deg_ref)
    a_ref[0] = t[:, :H]
    up_ref[0] = t[:, H:] * dinv[:, None]


def _tc_mid_body(a_ref, s_ref, deg_ref, w_ref, b_ref, a_out, up_out):
    dinv = _dinv_of(deg_ref)
    h = _elu(a_ref[0] - dinv[:, None] * s_ref[0] + b_ref[...])
    t = jnp.dot(h, w_ref[...], preferred_element_type=jnp.float32)
    a_out[0] = t[:, :H]
    up_out[0] = t[:, H:] * dinv[:, None]


def _tc4_body(a_ref, s_ref, deg_ref, b_ref, wl_ref, bl_ref, o_ref):
    i = pl.program_id(0)
    dinv = _dinv_of(deg_ref)
    h = _elu(a_ref[0] - dinv[:, None] * s_ref[0] + b_ref[...])
    pooled = jnp.sum(h, axis=0, keepdims=True) / float(N)
    out = jnp.dot(pooled, wl_ref[...],
                  preferred_element_type=jnp.float32) + bl_ref[...]
    m = jnp.max(out, axis=1, keepdims=True)
    z = out - m
    lse = jnp.log(jnp.sum(jnp.exp(z), axis=1, keepdims=True))
    o_ref[pl.ds(i, 1), :] = z - lse


_deg_spec = pl.BlockSpec((NSC, 1, N), lambda i: (0, 0, 0))
_nf_spec = pl.BlockSpec((1, N, H), lambda i: (i, 0, 0))
_b_spec = pl.BlockSpec((1, H), lambda i: (0, 0))

_tc1 = pl.pallas_call(
    _tc1_body,
    grid=(B,),
    in_specs=[
        pl.BlockSpec((1, C, N), lambda i: (i, 0, 0)),
        pl.BlockSpec((C, 2 * H), lambda i: (0, 0)),
        _deg_spec,
    ],
    out_specs=[_nf_spec, _nf_spec],
    out_shape=[
        jax.ShapeDtypeStruct((B, N, H), jnp.float32),
        jax.ShapeDtypeStruct((B, N, H), jnp.float32),
    ],
)

_tc_mid = pl.pallas_call(
    _tc_mid_body,
    grid=(B,),
    in_specs=[
        _nf_spec,
        _nf_spec,
        _deg_spec,
        pl.BlockSpec((H, 2 * H), lambda i: (0, 0)),
        _b_spec,
    ],
    out_specs=[_nf_spec, _nf_spec],
    out_shape=[
        jax.ShapeDtypeStruct((B, N, H), jnp.float32),
        jax.ShapeDtypeStruct((B, N, H), jnp.float32),
    ],
)

_tc4 = pl.pallas_call(
    _tc4_body,
    grid=(B,),
    in_specs=[
        _nf_spec,
        _nf_spec,
        _deg_spec,
        _b_spec,
        pl.BlockSpec((H, NCLS), lambda i: (0, 0)),
        pl.BlockSpec((1, NCLS), lambda i: (0, 0)),
    ],
    out_specs=pl.BlockSpec((B, NCLS), lambda i: (0, 0)),
    out_shape=jax.ShapeDtypeStruct((B, NCLS), jnp.float32),
)


def kernel(x, dummy, edge_index, W1_0, W1_1, b1, W2_0, W2_1, b2,
           W3_0, W3_1, b3, Wl, bl):
    row = jnp.pad(edge_index[0], (0, EP - E)).reshape(EPR, 128)
    col = jnp.pad(edge_index[1], (0, EP - E)).reshape(EPR, 128)

    deg, rg, cg = _build_sc_pre()(row, col)
    _sc_prop = _build_sc_prop()

    w1 = jnp.concatenate([W1_0, W1_1], axis=1)
    w2 = jnp.concatenate([W2_0, W2_1], axis=1)
    w3 = jnp.concatenate([W3_0, W3_1], axis=1)

    deg3 = deg[:, :N].reshape(NSC, 1, N)

    a1, u1 = _tc1(x, w1, deg3)
    s1 = _sc_prop(u1.reshape(BN, H), rg, cg)
    a2, u2 = _tc_mid(a1, s1.reshape(B, N, H), deg3, w2, b1.reshape(1, H))
    s2 = _sc_prop(u2.reshape(BN, H), rg, cg)
    a3, u3 = _tc_mid(a2, s2.reshape(B, N, H), deg3, w3, b2.reshape(1, H))
    s3 = _sc_prop(u3.reshape(BN, H), rg, cg)
    return _tc4(a3, s3.reshape(B, N, H), deg3, b3.reshape(1, H),
                Wl, bl.reshape(1, NCLS))


# speedup vs baseline: 107.4289x; 1.6371x over previous
import functools

import jax
import jax.numpy as jnp
from jax import lax
from jax.experimental import pallas as pl
from jax.experimental.pallas import tpu as pltpu
from jax.experimental.pallas import tpu_sc as plsc

B, C, N, E = 4, 128, 10000, 160000
H = 16
NCLS = 10
BN = B * N
TRASH = BN
LACC = 20480
LTRASH = 2 * N
DEG_TRASH = N
DEGP = 10240
EP = 163840
EPR = EP // 128
NSC, NTEC = 2, 16
NW = NSC * NTEC
W_CH = EPR // NW
W_EXP = EPR // 8
DEG_T = DEGP // NTEC
LACC_T = LACC // NTEC
OUT_T = (2 * N) // NTEC
RING = 8

NPR = N // 8
BLK = 400
NBLK = BN // BLK
BPG = N // BLK

@functools.cache
def _build_sc_pre():
  mesh = plsc.VectorSubcoreMesh(
      core_axis_name="c", subcore_axis_name="s",
      num_cores=NSC, num_subcores=NTEC)

  @functools.partial(
      pl.kernel,
      out_type=(
          jax.ShapeDtypeStruct((NSC, 1, DEGP), jnp.float32),
          jax.ShapeDtypeStruct((B, EPR, 128), jnp.int32),
          jax.ShapeDtypeStruct((B, EPR, 128), jnp.int32),
      ),
      mesh=mesh,
      compiler_params=pltpu.CompilerParams(use_tc_tiling_on_sc=False),
      scratch_types=[
          pltpu.VMEM((W_EXP, 128), jnp.int32),
          pltpu.VMEM((W_EXP, 128), jnp.int32),
          pltpu.VMEM((W_EXP, 128), jnp.int32),
          pltpu.VMEM((W_EXP, 128), jnp.int32),
          pltpu.VMEM((W_CH, 128), jnp.int32),
          pltpu.VMEM((128,), jnp.float32),
          pltpu.VMEM((DEG_T,), jnp.float32),
          pltpu.VMEM_SHARED((DEGP,), jnp.float32),
          pltpu.SemaphoreType.DMA,
      ],
  )
  def _sc_pre(row_hbm, col_hbm, deg_out, rg_out, cg_out,
              rbuf, cbuf, rgbuf, cgbuf, dix, ones_v, zeros_v, deg_acc, dsem):
    c = lax.axis_index("c")
    s = lax.axis_index("s")
    gwid = c * NTEC + s

    def _fill(i, _):
      ones_v[pl.ds(i * 16, 16)] = jnp.full((16,), 1.0, jnp.float32)
      return 0
    lax.fori_loop(0, 8, _fill, 0)

    def _zfill(i, _):
      zeros_v[pl.ds(i * 16, 16)] = jnp.zeros((16,), jnp.float32)
      return 0
    lax.fori_loop(0, DEG_T // 16, _zfill, 0)

    pltpu.sync_copy(zeros_v, deg_acc.at[pl.ds(s * DEG_T, DEG_T)])
    plsc.subcore_barrier()

    pltpu.sync_copy(row_hbm.at[pl.ds(gwid * W_CH, W_CH)], rbuf.at[pl.ds(0, W_CH)])
    pltpu.sync_copy(col_hbm.at[pl.ds(gwid * W_CH, W_CH)], cbuf.at[pl.ds(0, W_CH)])

    def _deg_chunk(j, _):
      for l in range(8):
        rv = rbuf[j, pl.ds(l * 16, 16)]
        cv = cbuf[j, pl.ds(l * 16, 16)]
        dix[j, pl.ds(l * 16, 16)] = jnp.where(
            rv == cv, jnp.full((16,), DEG_TRASH, jnp.int32), rv)
      pltpu.async_copy(ones_v, deg_acc.at[dix.at[j]], dsem, add=True)
      return 0
    lax.fori_loop(0, W_CH, _deg_chunk, 0)

    def _deg_drain(j, _):
      pltpu.make_async_copy(ones_v, deg_acc.at[dix.at[j]], dsem).wait()
      return 0
    lax.fori_loop(0, W_CH, _deg_drain, 0)

    plsc.subcore_barrier()
    pltpu.sync_copy(deg_acc.at[pl.ds(s * DEG_T, DEG_T)],
                    deg_out.at[c, 0, pl.ds(s * DEG_T, DEG_T)])

    b = gwid // 8
    w8 = gwid % 8
    lofs = (b % 2) * N
    pltpu.sync_copy(row_hbm.at[pl.ds(w8 * W_EXP, W_EXP)], rbuf)
    pltpu.sync_copy(col_hbm.at[pl.ds(w8 * W_EXP, W_EXP)], cbuf)

    def _exp_chunk(j, _):
      for l in range(8):
        rv = rbuf[j, pl.ds(l * 16, 16)]
        cv = cbuf[j, pl.ds(l * 16, 16)]
        rgbuf[j, pl.ds(l * 16, 16)] = rv + lofs
        cgbuf[j, pl.ds(l * 16, 16)] = jnp.where(
            rv == cv, jnp.full((16,), LTRASH, jnp.int32), cv + lofs)
      return 0
    lax.fori_loop(0, W_EXP, _exp_chunk, 0)

    pltpu.sync_copy(rgbuf, rg_out.at[b, pl.ds(w8 * W_EXP, W_EXP)])
    pltpu.sync_copy(cgbuf, cg_out.at[b, pl.ds(w8 * W_EXP, W_EXP)])

  return _sc_pre


@functools.cache
def _build_sc_prop():
  mesh = plsc.VectorSubcoreMesh(
      core_axis_name="c", subcore_axis_name="s",
      num_cores=NSC, num_subcores=NTEC)

  @functools.partial(
      pl.kernel,
      out_type=jax.ShapeDtypeStruct((BN, H), jnp.float32),
      mesh=mesh,
      compiler_params=pltpu.CompilerParams(use_tc_tiling_on_sc=False),
      scratch_types=[
          pltpu.VMEM((W_EXP, 128), jnp.int32),
          pltpu.VMEM((W_EXP, 128), jnp.int32),
          pltpu.VMEM((RING * 128, H), jnp.float32),
          pltpu.VMEM((128, H), jnp.float32),
          pltpu.VMEM_SHARED((LACC, H), jnp.float32),
          pltpu.VMEM_SHARED((2 * N, H), jnp.float32),
          [pltpu.SemaphoreType.DMA] * RING,
          [pltpu.SemaphoreType.DMA] * RING,
      ],
  )
  def _sc_prop(u_hbm, rg_hbm, cg_hbm, out_hbm,
               rix, cix, rows_v, zeros_v, acc, u_s, gsem, ssem):
    c = lax.axis_index("c")
    s = lax.axis_index("s")
    gwid = c * NTEC + s
    b = gwid // 8
    w8 = gwid % 8

    ixd = [
        pltpu.async_copy(rg_hbm.at[b, pl.ds(w8 * W_EXP, W_EXP)], rix, gsem[0]),
        pltpu.async_copy(cg_hbm.at[b, pl.ds(w8 * W_EXP, W_EXP)], cix, gsem[1]),
        pltpu.async_copy(u_hbm.at[pl.ds(c * 2 * N + s * OUT_T, OUT_T)],
                         u_s.at[pl.ds(s * OUT_T, OUT_T)], gsem[2]),
    ]

    def _zfill(i, _):
      zeros_v[i, :] = jnp.zeros((H,), jnp.float32)
      return 0
    lax.fori_loop(0, 128, _zfill, 0)

    def _zacc(k, _):
      pltpu.async_copy(zeros_v, acc.at[pl.ds(s * LACC_T + k * 128, 128)], ssem[0])
      return 0
    lax.fori_loop(0, LACC_T // 128, _zacc, 0)

    def _zdrain(k, _):
      pltpu.make_async_copy(zeros_v, acc.at[pl.ds(s * LACC_T + k * 128, 128)],
                            ssem[0]).wait()
      return 0
    lax.fori_loop(0, LACC_T // 128, _zdrain, 0)
    for d in ixd:
      d.wait()
    plsc.subcore_barrier()

    def _slot(r):
      return rows_v.at[pl.ds(r * 128, 128)]

    for r in range(RING):
      pltpu.async_copy(u_s.at[rix.at[r]], _slot(r), gsem[r])

    def _round(k, _):
      js = [k * RING + r for r in range(RING)]
      for r in range(RING):
        pltpu.make_async_copy(u_s.at[rix.at[js[r]]], _slot(r), gsem[r]).wait()
        pltpu.async_copy(_slot(r), acc.at[cix.at[js[r]]], ssem[r], add=True)
      for r in range(RING):
        pltpu.make_async_copy(_slot(r), acc.at[cix.at[js[r]]], ssem[r]).wait()

        @pl.when(k < W_EXP // RING - 1)
        def _():
          pltpu.async_copy(u_s.at[rix.at[js[r] + RING]], _slot(r), gsem[r])
      return 0
    lax.fori_loop(0, W_EXP // RING, _round, 0)

    plsc.subcore_barrier()
    pltpu.sync_copy(acc.at[pl.ds(s * OUT_T, OUT_T)],
                    out_hbm.at[pl.ds(c * 2 * N + s * OUT_T, OUT_T)])

  return _sc_prop


def _elu(v):
    return jnp.where(v > 0, v, jnp.exp(jnp.minimum(v, 0.0)) - 1.0)


def _dinv8_of(deg_ref, e_ref):
    d = deg_ref[0] + deg_ref[1]
    dv = jnp.where(d > 0, lax.rsqrt(jnp.where(d > 0, d, 1.0)), 0.0)
    return jnp.dot(dv[:NPR], e_ref[...],
                   preferred_element_type=jnp.float32)


def _tc1_body(x_ref, w0_ref, w1_ref, deg_ref, e_ref, a_ref, up_ref):
    xpk = x_ref[0]
    dpk = _dinv8_of(deg_ref, e_ref)
    a_ref[0] = jnp.dot(xpk, w0_ref[...], preferred_element_type=jnp.float32)
    up_ref[0] = dpk * jnp.dot(xpk, w1_ref[...],
                              preferred_element_type=jnp.float32)


def _tc_mid_body(a_ref, s_ref, deg_ref, e_ref, w0_ref, w1_ref, b_ref,
                 a_out, up_out):
    dpk = _dinv8_of(deg_ref, e_ref)
    h = _elu(a_ref[0] - dpk * s_ref[0] + b_ref[...])
    a_out[0] = jnp.dot(h, w0_ref[...], preferred_element_type=jnp.float32)
    up_out[0] = jnp.dot(dpk * h, w1_ref[...],
                        preferred_element_type=jnp.float32)


def _tc4_body(a_ref, s_ref, deg_ref, e_ref, f_ref, b_ref, wl_ref, bl_ref,
              o_ref):
    i = pl.program_id(0)
    dpk = _dinv8_of(deg_ref, e_ref)
    h = _elu(a_ref[0] - dpk * s_ref[0] + b_ref[...])
    v = jnp.sum(h, axis=0, keepdims=True)
    pooled = jnp.dot(v, f_ref[...],
                     preferred_element_type=jnp.float32) / float(N)
    out = jnp.dot(pooled, wl_ref[...],
                  preferred_element_type=jnp.float32) + bl_ref[...]
    m = jnp.max(out, axis=1, keepdims=True)
    z = out - m
    lse = jnp.log(jnp.sum(jnp.exp(z), axis=1, keepdims=True))
    o_ref[pl.ds(i, 1), :] = z - lse


_deg8_spec = pl.BlockSpec((NSC, DEGP // 8, 8), lambda i: (0, 0, 0))
_pk_spec = pl.BlockSpec((1, NPR, 128), lambda i: (i, 0, 0))
_nf_spec = pl.BlockSpec((1, N, H), lambda i: (i, 0, 0))
_bpk_spec = pl.BlockSpec((1, 128), lambda i: (0, 0))
_e_spec = pl.BlockSpec((8, 128), lambda i: (0, 0))

_tc1 = pl.pallas_call(
    _tc1_body,
    grid=(B,),
    in_specs=[
        pl.BlockSpec((1, NPR, 8 * C), lambda i: (i, 0, 0)),
        pl.BlockSpec((8 * C, 128), lambda i: (0, 0)),
        pl.BlockSpec((8 * C, 128), lambda i: (0, 0)),
        _deg8_spec,
        _e_spec,
    ],
    out_specs=[_pk_spec, _pk_spec],
    out_shape=[
        jax.ShapeDtypeStruct((B, NPR, 128), jnp.float32),
        jax.ShapeDtypeStruct((B, NPR, 128), jnp.float32),
    ],
)

_tc_mid = pl.pallas_call(
    _tc_mid_body,
    grid=(B,),
    in_specs=[
        _pk_spec,
        _pk_spec,
        _deg8_spec,
        _e_spec,
        pl.BlockSpec((128, 128), lambda i: (0, 0)),
        pl.BlockSpec((128, 128), lambda i: (0, 0)),
        _bpk_spec,
    ],
    out_specs=[_pk_spec, _pk_spec],
    out_shape=[
        jax.ShapeDtypeStruct((B, NPR, 128), jnp.float32),
        jax.ShapeDtypeStruct((B, NPR, 128), jnp.float32),
    ],
)

_tc4 = pl.pallas_call(
    _tc4_body,
    grid=(B,),
    in_specs=[
        _pk_spec,
        _pk_spec,
        _deg8_spec,
        _e_spec,
        pl.BlockSpec((128, H), lambda i: (0, 0)),
        _bpk_spec,
        pl.BlockSpec((H, NCLS), lambda i: (0, 0)),
        pl.BlockSpec((1, NCLS), lambda i: (0, 0)),
    ],
    out_specs=pl.BlockSpec((B, NCLS), lambda i: (0, 0)),
    out_shape=jax.ShapeDtypeStruct((B, NCLS), jnp.float32),
)


def kernel(x, dummy, edge_index, W1_0, W1_1, b1, W2_0, W2_1, b2,
           W3_0, W3_1, b3, Wl, bl):
    row = jnp.pad(edge_index[0], (0, EP - E)).reshape(EPR, 128)
    col = jnp.pad(edge_index[1], (0, EP - E)).reshape(EPR, 128)

    deg, rg, cg = _build_sc_pre()(row, col)
    _sc_prop = _build_sc_prop()
    f32 = jnp.float32

    deg8 = deg.reshape(NSC, DEGP // 8, 8)
    e16 = jnp.repeat(jnp.eye(8, dtype=f32), H, axis=1)
    f16 = jnp.tile(jnp.eye(H, dtype=f32), (8, 1))
    i8 = jnp.eye(8, dtype=f32)
    w1_0 = jnp.kron(i8, W1_0)
    w1_1 = jnp.kron(i8, W1_1)
    w2_0 = jnp.kron(i8, W2_0)
    w2_1 = jnp.kron(i8, W2_1)
    w3_0 = jnp.kron(i8, W3_0)
    w3_1 = jnp.kron(i8, W3_1)
    b1pk = jnp.tile(b1, 8).reshape(1, 128)
    b2pk = jnp.tile(b2, 8).reshape(1, 128)
    b3pk = jnp.tile(b3, 8).reshape(1, 128)

    xpk = jnp.moveaxis(x, 1, 2).reshape(B, NPR, 8 * C)
    a1, u1 = _tc1(xpk, w1_0, w1_1, deg8, e16)
    s1 = _sc_prop(u1.reshape(BN, H), rg, cg)
    a2, u2 = _tc_mid(a1, s1.reshape(B, NPR, 128),
                     deg8, e16, w2_0, w2_1, b1pk)
    s2 = _sc_prop(u2.reshape(BN, H), rg, cg)
    a3, u3 = _tc_mid(a2, s2.reshape(B, NPR, 128), deg8, e16, w3_0, w3_1, b2pk)
    s3 = _sc_prop(u3.reshape(BN, H), rg, cg)
    return _tc4(a3, s3.reshape(B, NPR, 128), deg8, e16, f16, b3pk,
                Wl, bl.reshape(1, NCLS))
